# baseline (speedup 1.0000x reference)
import jax
import jax.numpy as jnp
from jax import lax
from jax.experimental import pallas as pl
from jax.experimental.pallas import tpu as pltpu
from jax.experimental.pallas import tpu_sc as plsc

_TEMP = 1.0


def _proj_body(e_ref, wt_ref, b_ref, o_ref):
    o_ref[...] = (
        jnp.dot(e_ref[...], wt_ref[...], preferred_element_type=jnp.float32)
        + b_ref[...]
    )


def _project(e2, Wt, b2, N, D):
    rows = 2048
    return pl.pallas_call(
        _proj_body,
        grid=(N // rows,),
        in_specs=[
            pl.BlockSpec((rows, D), lambda i: (i, 0)),
            pl.BlockSpec((D, D), lambda i: (0, 0)),
            pl.BlockSpec((1, D), lambda i: (0, 0)),
        ],
        out_specs=pl.BlockSpec((rows, D), lambda i: (i, 0)),
        out_shape=jax.ShapeDtypeStruct((N, D), jnp.float32),
    )(e2, Wt, b2)


def _widen_body(t_ref, e_ref, o_ref):
    o_ref[:, 0:64] = lax.dot_general(
        t_ref[...], e_ref[...], (((0,), (0,)), ((), ())),
        preferred_element_type=jnp.float32)


def _widen_table(table_t, V, D):
    rows = 2048
    eye = jnp.eye(D, dtype=jnp.float32)
    return pl.pallas_call(
        _widen_body,
        grid=((V + rows - 1) // rows,),
        in_specs=[pl.BlockSpec((D, rows), lambda i: (0, i)),
                  pl.BlockSpec((D, D), lambda i: (0, 0))],
        out_specs=pl.BlockSpec((rows, 2 * D), lambda i: (i, 0)),
        out_shape=jax.ShapeDtypeStruct((V, 2 * D), jnp.float32),
    )(table_t, eye)


def _make_scores(N, D, NEG):
    NC, NS = 2, 16
    NW = NC * NS
    RW = N // NW
    R = 4
    NCH = RW // R
    L = 16
    JG = NEG // L
    IW = 72
    G = 1 + NEG
    mesh = plsc.VectorSubcoreMesh(
        core_axis_name="c", subcore_axis_name="s", num_cores=NC, num_subcores=NS
    )

    def body(x_hbm, idxc_hbm, table_hbm, pos_hbm, neg_hbm,
             idx_all, comb_v, x_v, nout_v, pout_all,
             sem_in0, sem_in1, sem_out0, sem_out1):
        wid = lax.axis_index("s") * NC + lax.axis_index("c")
        base = wid * RW
        pltpu.sync_copy(idxc_hbm.at[pl.ds(base * IW, RW * IW)], idx_all)

        sems_in = (sem_in0, sem_in1)
        sems_out = (sem_out0, sem_out1)
        lane = jnp.arange(L, dtype=jnp.int32)

        def in_copies(gg, b):
            lr = gg * R
            sem = sems_in[b]
            cps = [
                pltpu.make_async_copy(
                    x_hbm.at[pl.ds((base + lr) * D, R * D)], x_v.at[b], sem),
            ]
            for r in range(R):
                cps.append(pltpu.make_async_copy(
                    table_hbm.at[idx_all.at[pl.ds((lr + r) * IW, G)]],
                    comb_v.at[b, r, pl.ds(0, G)], sem))
            return cps

        def out_copies(gg, b):
            return [
                pltpu.make_async_copy(
                    nout_v.at[b],
                    neg_hbm.at[pl.ds((base + gg * R) * NEG, R * NEG)],
                    sems_out[b]),
            ]

        def fire(gg, b):
            for c in in_copies(gg, b):
                c.start()

        fire(0, 0)
        fire(1, 1)

        def tbody(t, pos_acc):
            for b in range(2):
                gg = t * 2 + b
                for c in in_copies(gg, b):
                    c.wait()

                @pl.when(gg >= 2)
                def _():
                    for c in out_copies(gg - 2, b):
                        c.wait()

                for r in range(R):
                    x0 = x_v[b, r * D + 0:r * D + 16]
                    x1 = x_v[b, r * D + 16:r * D + 32]
                    x2 = x_v[b, r * D + 32:r * D + 48]
                    x3 = x_v[b, r * D + 48:r * D + 64]
                    ps = (comb_v[b, r, 0, 0:16] * x0
                          + comb_v[b, r, 0, 16:32] * x1
                          + comb_v[b, r, 0, 32:48] * x2
                          + comb_v[b, r, 0, 48:64] * x3)
                    ln = 8 * (t % 2) + 4 * b + r
                    pos_acc = pos_acc + jnp.sum(ps) * jnp.where(
                        lane == ln, 1.0, 0.0).astype(jnp.float32)

                    bi = jnp.full((L,), b, jnp.int32)
                    ri = jnp.full((L,), r, jnp.int32)

                    def gbody(jg, c2, b=b, r=r, bi=bi, ri=ri,
                              x0=x0, x1=x1, x2=x2, x3=x3):
                        acc = jnp.zeros((L,), jnp.float32)
                        rb = jg * L + 1
                        for jj in range(L):
                            row_i = jnp.full((L,), rb + jj, jnp.int32)
                            a = (plsc.load_gather(
                                     comb_v, [bi, ri, row_i, lane]) * x0
                                 + plsc.load_gather(
                                     comb_v, [bi, ri, row_i, lane + 16]) * x1
                                 + plsc.load_gather(
                                     comb_v, [bi, ri, row_i, lane + 32]) * x2
                                 + plsc.load_gather(
                                     comb_v, [bi, ri, row_i, lane + 48]) * x3)
                            oh = jnp.where(lane == jj, 1.0, 0.0).astype(
                                jnp.float32)
                            acc = acc + jnp.sum(a) * oh
                        plsc.store_scatter(
                            nout_v, [bi, r * NEG + jg * L + lane], acc)
                        return c2

                    lax.fori_loop(0, JG, gbody, 0)

                for c in out_copies(gg, b):
                    c.start()

                @pl.when(gg + 2 < NCH)
                def _():
                    fire(gg + 2, b)

            @pl.when(t % 2 == 1)
            def _():
                o = (t // 2) * L + lane
                plsc.store_scatter(pout_all, [o // 128, o % 128], pos_acc)

            return jnp.where(t % 2 == 1, jnp.zeros((L,), jnp.float32),
                             pos_acc)

        lax.fori_loop(0, NCH // 2, tbody, jnp.zeros((L,), jnp.float32))
        for c in out_copies(NCH - 2, 0):
            c.wait()
        for c in out_copies(NCH - 1, 1):
            c.wait()
        pltpu.sync_copy(pout_all, pos_hbm.at[wid])

    return pl.kernel(
        body,
        out_type=[
            jax.ShapeDtypeStruct((NW, RW // 128, 128), jnp.float32),
            jax.ShapeDtypeStruct((N * NEG,), jnp.float32),
        ],
        mesh=mesh,
        compiler_params=pltpu.CompilerParams(
            needs_layout_passes=False, use_tc_tiling_on_sc=True),
        scratch_types=[
            pltpu.VMEM((RW * IW,), jnp.int32),
            pltpu.VMEM((2, R, IW, 2 * D), jnp.float32),
            pltpu.VMEM((2, R * D), jnp.float32),
            pltpu.VMEM((2, R * NEG), jnp.float32),
            pltpu.VMEM((RW // 128, 128), jnp.float32),
            pltpu.SemaphoreType.DMA,
            pltpu.SemaphoreType.DMA,
            pltpu.SemaphoreType.DMA,
            pltpu.SemaphoreType.DMA,
        ],
    )


def _make_loss(N, NEG):
    rows = 2048
    inv_t = 1.0 / _TEMP

    def body(pos_ref, neg_ref, logits_ref, loss_ref):
        i = pl.program_id(0)
        pos = pos_ref[...]
        neg = neg_ref[...]
        logits_ref[...] = jnp.concatenate([pos, neg], axis=1)
        sp = pos * inv_t
        sn = neg * inv_t
        m = jnp.maximum(sp, jnp.max(sn, axis=1, keepdims=True))
        lse = m + jnp.log(
            jnp.exp(sp - m) + jnp.sum(jnp.exp(sn - m), axis=1, keepdims=True))
        c = jnp.sum(lse - sp)

        @pl.when(i == 0)
        def _():
            loss_ref[0, 0] = 0.0

        loss_ref[0, 0] += c / N

    return pl.pallas_call(
        body,
        grid=(N // rows,),
        in_specs=[
            pl.BlockSpec((rows, 1), lambda i: (i, 0)),
            pl.BlockSpec((rows, NEG), lambda i: (i, 0)),
        ],
        out_specs=[
            pl.BlockSpec((rows, 1 + NEG), lambda i: (i, 0)),
            pl.BlockSpec((1, 1), lambda i: (0, 0), memory_space=pltpu.SMEM),
        ],
        out_shape=[
            jax.ShapeDtypeStruct((N, 1 + NEG), jnp.float32),
            jax.ShapeDtypeStruct((1, 1), jnp.float32),
        ],
    )


def kernel(embeds, labels, table, W, b, neg_samples):
    B, T, D = embeds.shape
    N = B * T
    V = table.shape[0]
    NEG = neg_samples.shape[1]

    e2 = embeds.reshape(N, D)
    x = _project(e2, W.T, b.reshape(1, D), N, D)
    table_w = _widen_table(table.T, V, D)
    idxc = jnp.concatenate(
        [labels.reshape(N, 1), neg_samples,
         jnp.zeros((N, 7), jnp.int32)], axis=1).reshape(-1)
    pos3, neg1 = _make_scores(N, D, NEG)(x.reshape(-1), idxc, table_w)
    pos = pos3.reshape(N, 1)
    neg = neg1.reshape(N, NEG)
    logits, loss = _make_loss(N, NEG)(pos, neg)
    return logits, loss.reshape(())


# speedup vs baseline: 1.2832x; 1.2832x over previous
import jax
import jax.numpy as jnp
from jax import lax
from jax.experimental import pallas as pl
from jax.experimental.pallas import tpu as pltpu
from jax.experimental.pallas import tpu_sc as plsc

_TEMP = 1.0


def _proj_body(e_ref, wt_ref, b_ref, o_ref):
    o_ref[...] = (
        jnp.dot(e_ref[...], wt_ref[...], preferred_element_type=jnp.float32)
        + b_ref[...]
    )


def _project(e2, Wt, b2, N, D):
    rows = 2048
    return pl.pallas_call(
        _proj_body,
        grid=(N // rows,),
        in_specs=[
            pl.BlockSpec((rows, D), lambda i: (i, 0)),
            pl.BlockSpec((D, D), lambda i: (0, 0)),
            pl.BlockSpec((1, D), lambda i: (0, 0)),
        ],
        out_specs=pl.BlockSpec((rows, D), lambda i: (i, 0)),
        out_shape=jax.ShapeDtypeStruct((N, D), jnp.float32),
    )(e2, Wt, b2)


def _widen_body(t_ref, e_ref, o_ref):
    o_ref[:, 0:64] = lax.dot_general(
        t_ref[...], e_ref[...], (((0,), (0,)), ((), ())),
        preferred_element_type=jnp.float32)


def _widen_table(table_t, V, D):
    rows = 8192
    eye = jnp.eye(D, dtype=jnp.float32)
    return pl.pallas_call(
        _widen_body,
        grid=((V + rows - 1) // rows,),
        in_specs=[pl.BlockSpec((D, rows), lambda i: (0, i)),
                  pl.BlockSpec((D, D), lambda i: (0, 0))],
        out_specs=pl.BlockSpec((rows, 2 * D), lambda i: (i, 0)),
        out_shape=jax.ShapeDtypeStruct((V, 2 * D), jnp.float32),
    )(table_t, eye)


def _make_scores(N, D, NEG):
    NC, NS = 2, 16
    NW = NC * NS
    RW = N // NW
    R = 4
    NCH = RW // R
    L = 16
    JG = NEG // L
    IW = 72
    G = 1 + NEG
    mesh = plsc.VectorSubcoreMesh(
        core_axis_name="c", subcore_axis_name="s", num_cores=NC, num_subcores=NS
    )

    def body(x_hbm, idxc_hbm, table_hbm, pos_hbm, neg_hbm,
             idx_all, comb_v, x_v, nout_v, pout_all,
             sem_in0, sem_in1, sem_out0, sem_out1):
        wid = lax.axis_index("s") * NC + lax.axis_index("c")
        base = wid * RW
        pltpu.sync_copy(idxc_hbm.at[pl.ds(base * IW, RW * IW)], idx_all)

        sems_in = (sem_in0, sem_in1)
        sems_out = (sem_out0, sem_out1)
        lane = jnp.arange(L, dtype=jnp.int32)

        def in_copies(gg, b):
            lr = gg * R
            sem = sems_in[b]
            cps = [
                pltpu.make_async_copy(
                    x_hbm.at[pl.ds((base + lr) * D, R * D)], x_v.at[b], sem),
            ]
            for r in range(R):
                cps.append(pltpu.make_async_copy(
                    table_hbm.at[idx_all.at[pl.ds((lr + r) * IW, G)]],
                    comb_v.at[b, r, pl.ds(0, G)], sem))
            return cps

        def out_copies(gg, b):
            return [
                pltpu.make_async_copy(
                    nout_v.at[b],
                    neg_hbm.at[pl.ds((base + gg * R) * NEG, R * NEG)],
                    sems_out[b]),
            ]

        def fire(gg, b):
            for c in in_copies(gg, b):
                c.start()

        fire(0, 0)
        fire(1, 1)

        def tbody(t, pos_acc):
            for b in range(2):
                gg = t * 2 + b
                for c in in_copies(gg, b):
                    c.wait()

                @pl.when(gg >= 2)
                def _():
                    for c in out_copies(gg - 2, b):
                        c.wait()

                for r in range(R):
                    x0 = x_v[b, r * D + 0:r * D + 16]
                    x1 = x_v[b, r * D + 16:r * D + 32]
                    x2 = x_v[b, r * D + 32:r * D + 48]
                    x3 = x_v[b, r * D + 48:r * D + 64]
                    ps = (comb_v[b, r, 0, 0:16] * x0
                          + comb_v[b, r, 0, 16:32] * x1
                          + comb_v[b, r, 0, 32:48] * x2
                          + comb_v[b, r, 0, 48:64] * x3)
                    ln = 8 * (t % 2) + 4 * b + r
                    pos_acc = pos_acc + jnp.sum(ps) * jnp.where(
                        lane == ln, 1.0, 0.0).astype(jnp.float32)

                    bi = jnp.full((L,), b, jnp.int32)
                    ri = jnp.full((L,), r, jnp.int32)

                    def gbody(jg, c2, b=b, r=r, bi=bi, ri=ri,
                              x0=x0, x1=x1, x2=x2, x3=x3):
                        acc = jnp.zeros((L,), jnp.float32)
                        rb = jg * L + 1
                        for jj in range(L):
                            row_i = jnp.full((L,), rb + jj, jnp.int32)
                            a = (plsc.load_gather(
                                     comb_v, [bi, ri, row_i, lane]) * x0
                                 + plsc.load_gather(
                                     comb_v, [bi, ri, row_i, lane + 16]) * x1
                                 + plsc.load_gather(
                                     comb_v, [bi, ri, row_i, lane + 32]) * x2
                                 + plsc.load_gather(
                                     comb_v, [bi, ri, row_i, lane + 48]) * x3)
                            oh = jnp.where(lane == jj, 1.0, 0.0).astype(
                                jnp.float32)
                            acc = acc + jnp.sum(a) * oh
                        plsc.store_scatter(
                            nout_v, [bi, r * NEG + jg * L + lane], acc)
                        return c2

                    lax.fori_loop(0, JG, gbody, 0)

                for c in out_copies(gg, b):
                    c.start()

                @pl.when(gg + 2 < NCH)
                def _():
                    fire(gg + 2, b)

            @pl.when(t % 2 == 1)
            def _():
                o = (t // 2) * L + lane
                plsc.store_scatter(pout_all, [o // 128, o % 128], pos_acc)

            return jnp.where(t % 2 == 1, jnp.zeros((L,), jnp.float32),
                             pos_acc)

        lax.fori_loop(0, NCH // 2, tbody, jnp.zeros((L,), jnp.float32))
        for c in out_copies(NCH - 2, 0):
            c.wait()
        for c in out_copies(NCH - 1, 1):
            c.wait()
        pltpu.sync_copy(pout_all, pos_hbm.at[wid])

    return pl.kernel(
        body,
        out_type=[
            jax.ShapeDtypeStruct((NW, RW // 128, 128), jnp.float32),
            jax.ShapeDtypeStruct((N * NEG,), jnp.float32),
        ],
        mesh=mesh,
        compiler_params=pltpu.CompilerParams(
            needs_layout_passes=False, use_tc_tiling_on_sc=True),
        scratch_types=[
            pltpu.VMEM((RW * IW,), jnp.int32),
            pltpu.VMEM((2, R, IW, 2 * D), jnp.float32),
            pltpu.VMEM((2, R * D), jnp.float32),
            pltpu.VMEM((2, R * NEG), jnp.float32),
            pltpu.VMEM((RW // 128, 128), jnp.float32),
            pltpu.SemaphoreType.DMA,
            pltpu.SemaphoreType.DMA,
            pltpu.SemaphoreType.DMA,
            pltpu.SemaphoreType.DMA,
        ],
    )


def _make_loss(N, NEG):
    rows = 2048
    inv_t = 1.0 / _TEMP

    def body(pos_ref, neg_ref, logits_ref, loss_ref):
        i = pl.program_id(0)
        pos = pos_ref[...]
        neg = neg_ref[...]
        logits_ref[...] = jnp.concatenate([pos, neg], axis=1)
        sp = pos * inv_t
        sn = neg * inv_t
        m = jnp.maximum(sp, jnp.max(sn, axis=1, keepdims=True))
        lse = m + jnp.log(
            jnp.exp(sp - m) + jnp.sum(jnp.exp(sn - m), axis=1, keepdims=True))
        c = jnp.sum(lse - sp)

        @pl.when(i == 0)
        def _():
            loss_ref[0, 0] = 0.0

        loss_ref[0, 0] += c / N

    return pl.pallas_call(
        body,
        grid=(N // rows,),
        in_specs=[
            pl.BlockSpec((rows, 1), lambda i: (i, 0)),
            pl.BlockSpec((rows, NEG), lambda i: (i, 0)),
        ],
        out_specs=[
            pl.BlockSpec((rows, 1 + NEG), lambda i: (i, 0)),
            pl.BlockSpec((1, 1), lambda i: (0, 0), memory_space=pltpu.SMEM),
        ],
        out_shape=[
            jax.ShapeDtypeStruct((N, 1 + NEG), jnp.float32),
            jax.ShapeDtypeStruct((1, 1), jnp.float32),
        ],
    )


def kernel(embeds, labels, table, W, b, neg_samples):
    B, T, D = embeds.shape
    N = B * T
    V = table.shape[0]
    NEG = neg_samples.shape[1]

    e2 = embeds.reshape(N, D)
    x = _project(e2, W.T, b.reshape(1, D), N, D)
    table_w = _widen_table(table.T, V, D)
    idxc = jnp.concatenate(
        [labels.reshape(N, 1), neg_samples,
         jnp.zeros((N, 7), jnp.int32)], axis=1).reshape(-1)
    pos3, neg1 = _make_scores(N, D, NEG)(x.reshape(-1), idxc, table_w)
    pos = pos3.reshape(N, 1)
    neg = neg1.reshape(N, NEG)
    logits, loss = _make_loss(N, NEG)(pos, neg)
    return logits, loss.reshape(())


# speedup vs baseline: 1.3271x; 1.0342x over previous
import jax
import jax.numpy as jnp
from jax import lax
from jax.experimental import pallas as pl
from jax.experimental.pallas import tpu as pltpu
from jax.experimental.pallas import tpu_sc as plsc

_TEMP = 1.0


def _proj_body(e_ref, wt_ref, b_ref, o_ref):
    o_ref[...] = (
        jnp.dot(e_ref[...], wt_ref[...], preferred_element_type=jnp.float32)
        + b_ref[...]
    )


def _project(e2, Wt, b2, N, D):
    rows = 2048
    return pl.pallas_call(
        _proj_body,
        grid=(N // rows,),
        in_specs=[
            pl.BlockSpec((rows, D), lambda i: (i, 0)),
            pl.BlockSpec((D, D), lambda i: (0, 0)),
            pl.BlockSpec((1, D), lambda i: (0, 0)),
        ],
        out_specs=pl.BlockSpec((rows, D), lambda i: (i, 0)),
        out_shape=jax.ShapeDtypeStruct((N, D), jnp.float32),
    )(e2, Wt, b2)


def _widen_body(t_ref, e_ref, o_ref):
    o_ref[:, 0:64] = lax.dot_general(
        t_ref[...], e_ref[...], (((0,), (0,)), ((), ())),
        preferred_element_type=jnp.float32)


def _widen_table(table_t, V, D):
    rows = 16384
    eye = jnp.eye(D, dtype=jnp.float32)
    return pl.pallas_call(
        _widen_body,
        grid=((V + rows - 1) // rows,),
        in_specs=[pl.BlockSpec((D, rows), lambda i: (0, i)),
                  pl.BlockSpec((D, D), lambda i: (0, 0))],
        out_specs=pl.BlockSpec((rows, 2 * D), lambda i: (i, 0)),
        out_shape=jax.ShapeDtypeStruct((V, 2 * D), jnp.float32),
    )(table_t, eye)


def _make_scores(N, D, NEG):
    NC, NS = 2, 16
    NW = NC * NS
    RW = N // NW
    R = 4
    NCH = RW // R
    L = 16
    JG = NEG // L
    IW = 72
    G = 1 + NEG
    mesh = plsc.VectorSubcoreMesh(
        core_axis_name="c", subcore_axis_name="s", num_cores=NC, num_subcores=NS
    )

    def body(x_hbm, idxc_hbm, table_hbm, pos_hbm, neg_hbm,
             idx_all, comb_v, x_v, nout_v, pout_all,
             sem_in0, sem_in1, sem_out0, sem_out1):
        wid = lax.axis_index("s") * NC + lax.axis_index("c")
        base = wid * RW
        pltpu.sync_copy(idxc_hbm.at[pl.ds(base * IW, RW * IW)], idx_all)

        sems_in = (sem_in0, sem_in1)
        sems_out = (sem_out0, sem_out1)
        lane = jnp.arange(L, dtype=jnp.int32)

        def in_copies(gg, b):
            lr = gg * R
            sem = sems_in[b]
            cps = [
                pltpu.make_async_copy(
                    x_hbm.at[pl.ds((base + lr) * D, R * D)], x_v.at[b], sem),
            ]
            for r in range(R):
                cps.append(pltpu.make_async_copy(
                    table_hbm.at[idx_all.at[pl.ds((lr + r) * IW, G)]],
                    comb_v.at[b, r, pl.ds(0, G)], sem))
            return cps

        def out_copies(gg, b):
            return [
                pltpu.make_async_copy(
                    nout_v.at[b],
                    neg_hbm.at[pl.ds((base + gg * R) * NEG, R * NEG)],
                    sems_out[b]),
            ]

        def fire(gg, b):
            for c in in_copies(gg, b):
                c.start()

        fire(0, 0)
        fire(1, 1)

        def tbody(t, pos_acc):
            for b in range(2):
                gg = t * 2 + b
                for c in in_copies(gg, b):
                    c.wait()

                @pl.when(gg >= 2)
                def _():
                    for c in out_copies(gg - 2, b):
                        c.wait()

                for r in range(R):
                    x0 = x_v[b, r * D + 0:r * D + 16]
                    x1 = x_v[b, r * D + 16:r * D + 32]
                    x2 = x_v[b, r * D + 32:r * D + 48]
                    x3 = x_v[b, r * D + 48:r * D + 64]
                    ps = (comb_v[b, r, 0, 0:16] * x0
                          + comb_v[b, r, 0, 16:32] * x1
                          + comb_v[b, r, 0, 32:48] * x2
                          + comb_v[b, r, 0, 48:64] * x3)
                    ln = 8 * (t % 2) + 4 * b + r
                    pos_acc = pos_acc + jnp.sum(ps) * jnp.where(
                        lane == ln, 1.0, 0.0).astype(jnp.float32)

                    bi = jnp.full((L,), b, jnp.int32)
                    ri = jnp.full((L,), r, jnp.int32)

                    def gbody(jg, c2, b=b, r=r, bi=bi, ri=ri,
                              x0=x0, x1=x1, x2=x2, x3=x3):
                        acc = jnp.zeros((L,), jnp.float32)
                        rb = jg * L + 1
                        for jj in range(L):
                            row_i = jnp.full((L,), rb + jj, jnp.int32)
                            a = (plsc.load_gather(
                                     comb_v, [bi, ri, row_i, lane]) * x0
                                 + plsc.load_gather(
                                     comb_v, [bi, ri, row_i, lane + 16]) * x1
                                 + plsc.load_gather(
                                     comb_v, [bi, ri, row_i, lane + 32]) * x2
                                 + plsc.load_gather(
                                     comb_v, [bi, ri, row_i, lane + 48]) * x3)
                            oh = jnp.where(lane == jj, 1.0, 0.0).astype(
                                jnp.float32)
                            acc = acc + jnp.sum(a) * oh
                        plsc.store_scatter(
                            nout_v, [bi, r * NEG + jg * L + lane], acc)
                        return c2

                    lax.fori_loop(0, JG, gbody, 0)

                for c in out_copies(gg, b):
                    c.start()

                @pl.when(gg + 2 < NCH)
                def _():
                    fire(gg + 2, b)

            @pl.when(t % 2 == 1)
            def _():
                o = (t // 2) * L + lane
                plsc.store_scatter(pout_all, [o // 128, o % 128], pos_acc)

            return jnp.where(t % 2 == 1, jnp.zeros((L,), jnp.float32),
                             pos_acc)

        lax.fori_loop(0, NCH // 2, tbody, jnp.zeros((L,), jnp.float32))
        for c in out_copies(NCH - 2, 0):
            c.wait()
        for c in out_copies(NCH - 1, 1):
            c.wait()
        pltpu.sync_copy(pout_all, pos_hbm.at[wid])

    return pl.kernel(
        body,
        out_type=[
            jax.ShapeDtypeStruct((NW, RW // 128, 128), jnp.float32),
            jax.ShapeDtypeStruct((N * NEG,), jnp.float32),
        ],
        mesh=mesh,
        compiler_params=pltpu.CompilerParams(
            needs_layout_passes=False, use_tc_tiling_on_sc=True),
        scratch_types=[
            pltpu.VMEM((RW * IW,), jnp.int32),
            pltpu.VMEM((2, R, IW, 2 * D), jnp.float32),
            pltpu.VMEM((2, R * D), jnp.float32),
            pltpu.VMEM((2, R * NEG), jnp.float32),
            pltpu.VMEM((RW // 128, 128), jnp.float32),
            pltpu.SemaphoreType.DMA,
            pltpu.SemaphoreType.DMA,
            pltpu.SemaphoreType.DMA,
            pltpu.SemaphoreType.DMA,
        ],
    )


def _make_loss(N, NEG):
    rows = 2048
    inv_t = 1.0 / _TEMP

    def body(pos_ref, neg_ref, logits_ref, loss_ref):
        i = pl.program_id(0)
        pos = pos_ref[...]
        neg = neg_ref[...]
        logits_ref[...] = jnp.concatenate([pos, neg], axis=1)
        sp = pos * inv_t
        sn = neg * inv_t
        m = jnp.maximum(sp, jnp.max(sn, axis=1, keepdims=True))
        lse = m + jnp.log(
            jnp.exp(sp - m) + jnp.sum(jnp.exp(sn - m), axis=1, keepdims=True))
        c = jnp.sum(lse - sp)

        @pl.when(i == 0)
        def _():
            loss_ref[0, 0] = 0.0

        loss_ref[0, 0] += c / N

    return pl.pallas_call(
        body,
        grid=(N // rows,),
        in_specs=[
            pl.BlockSpec((rows, 1), lambda i: (i, 0)),
            pl.BlockSpec((rows, NEG), lambda i: (i, 0)),
        ],
        out_specs=[
            pl.BlockSpec((rows, 1 + NEG), lambda i: (i, 0)),
            pl.BlockSpec((1, 1), lambda i: (0, 0), memory_space=pltpu.SMEM),
        ],
        out_shape=[
            jax.ShapeDtypeStruct((N, 1 + NEG), jnp.float32),
            jax.ShapeDtypeStruct((1, 1), jnp.float32),
        ],
    )


def kernel(embeds, labels, table, W, b, neg_samples):
    B, T, D = embeds.shape
    N = B * T
    V = table.shape[0]
    NEG = neg_samples.shape[1]

    e2 = embeds.reshape(N, D)
    x = _project(e2, W.T, b.reshape(1, D), N, D)
    table_w = _widen_table(table.T, V, D)
    idxc = jnp.concatenate(
        [labels.reshape(N, 1), neg_samples,
         jnp.zeros((N, 7), jnp.int32)], axis=1).reshape(-1)
    pos3, neg1 = _make_scores(N, D, NEG)(x.reshape(-1), idxc, table_w)
    pos = pos3.reshape(N, 1)
    neg = neg1.reshape(N, NEG)
    logits, loss = _make_loss(N, NEG)(pos, neg)
    return logits, loss.reshape(())


# speedup vs baseline: 1.4253x; 1.0739x over previous
import jax
import jax.numpy as jnp
from jax import lax
from jax.experimental import pallas as pl
from jax.experimental.pallas import tpu as pltpu
from jax.experimental.pallas import tpu_sc as plsc

_TEMP = 1.0


def _proj_body(e_ref, wt_ref, b_ref, o_ref):
    o_ref[...] = (
        jnp.dot(e_ref[...], wt_ref[...], preferred_element_type=jnp.float32)
        + b_ref[...]
    )


def _project(e2, Wt, b2, N, D):
    rows = 2048
    return pl.pallas_call(
        _proj_body,
        grid=(N // rows,),
        in_specs=[
            pl.BlockSpec((rows, D), lambda i: (i, 0)),
            pl.BlockSpec((D, D), lambda i: (0, 0)),
            pl.BlockSpec((1, D), lambda i: (0, 0)),
        ],
        out_specs=pl.BlockSpec((rows, D), lambda i: (i, 0)),
        out_shape=jax.ShapeDtypeStruct((N, D), jnp.float32),
    )(e2, Wt, b2)


def _widen_body(t_ref, e_ref, o_ref):
    o_ref[:, 0:64] = lax.dot_general(
        t_ref[...], e_ref[...], (((0,), (0,)), ((), ())),
        preferred_element_type=jnp.float32)


def _widen_table(table_t, V, D):
    rows = 16384
    eye = jnp.eye(D, dtype=jnp.float32)
    return pl.pallas_call(
        _widen_body,
        grid=((V + rows - 1) // rows,),
        in_specs=[pl.BlockSpec((D, rows), lambda i: (0, i)),
                  pl.BlockSpec((D, D), lambda i: (0, 0))],
        out_specs=pl.BlockSpec((rows, 2 * D), lambda i: (i, 0)),
        out_shape=jax.ShapeDtypeStruct((V, 2 * D), jnp.float32),
    )(table_t, eye)


def _make_scores(N, D, NEG):
    NC, NS = 2, 16
    NW = NC * NS
    RW = N // NW
    R = 4
    NCH = RW // R
    L = 16
    JG = NEG // L
    IW = 72
    G = 1 + NEG
    mesh = plsc.VectorSubcoreMesh(
        core_axis_name="c", subcore_axis_name="s", num_cores=NC, num_subcores=NS
    )

    NB = 3

    def body(x_hbm, idxc_hbm, table_hbm, pos_hbm, neg_hbm,
             idx_v, comb_v, x_v, nout_v, pout_all,
             sem_idx, sem_in, sem_out, sem_spare):
        wid = lax.axis_index("s") * NC + lax.axis_index("c")
        base = wid * RW
        lane = jnp.arange(L, dtype=jnp.int32)

        def idx_copy(c):
            return pltpu.make_async_copy(
                idxc_hbm.at[pl.ds((base + c * R) * IW, R * IW)],
                idx_v.at[pl.ds((c % NB) * R * IW, R * IW)], sem_idx)

        def in_copies(c):
            b = c % NB
            cps = [
                pltpu.make_async_copy(
                    x_hbm.at[pl.ds((base + c * R) * D, R * D)],
                    x_v.at[pl.ds(b * R * D, R * D)], sem_in),
            ]
            for r in range(R):
                cps.append(pltpu.make_async_copy(
                    table_hbm.at[idx_v.at[pl.ds(b * R * IW + r * IW, G)]],
                    comb_v.at[b, r, pl.ds(0, G)], sem_in))
            return cps

        def out_copy(c):
            return pltpu.make_async_copy(
                nout_v.at[pl.ds((c % NB) * R * NEG, R * NEG)],
                neg_hbm.at[pl.ds((base + c * R) * NEG, R * NEG)], sem_out)

        for c in range(2):
            idx_copy(c).start()
            idx_copy(c).wait()
            for cp in in_copies(c):
                cp.start()
        idx_copy(2).start()

        def tbody(t, carry):
            b = t % NB
            bi = jnp.full((L,), b, jnp.int32)
            for c in in_copies(t):
                c.wait()

            @pl.when(t >= NB)
            def _():
                out_copy(t - NB).wait()

            for r in range(R):
                ri = jnp.full((L,), r, jnp.int32)

                def ld(row_vec, k):
                    return plsc.load_gather(
                        comb_v, [bi, ri, row_vec, lane + 16 * k])

                def xld(k, r=r):
                    return plsc.load_gather(
                        x_v, [bi * (R * D) + r * D + 16 * k + lane])

                x0, x1, x2, x3 = xld(0), xld(1), xld(2), xld(3)
                zero = jnp.zeros((L,), jnp.int32)
                ps = (ld(zero, 0) * x0 + ld(zero, 1) * x1
                      + ld(zero, 2) * x2 + ld(zero, 3) * x3)
                o = t * R + r
                plsc.store_scatter(
                    pout_all, [jnp.full((L,), o // 128, jnp.int32),
                               jnp.full((L,), o % 128, jnp.int32)],
                    jnp.full((L,), jnp.sum(ps), jnp.float32),
                    mask=lane == 0)

                def gbody(jg, c2, r=r, bi=bi, ri=ri,
                          x0=x0, x1=x1, x2=x2, x3=x3):
                    acc = jnp.zeros((L,), jnp.float32)
                    rb = jg * L + 1
                    for jj in range(L):
                        row_i = jnp.full((L,), rb + jj, jnp.int32)
                        a = (ld(row_i, 0) * x0 + ld(row_i, 1) * x1
                             + ld(row_i, 2) * x2 + ld(row_i, 3) * x3)
                        oh = jnp.where(lane == jj, 1.0, 0.0).astype(
                            jnp.float32)
                        acc = acc + jnp.sum(a) * oh
                    plsc.store_scatter(
                        nout_v,
                        [bi * (R * NEG) + r * NEG + jg * L + lane], acc)
                    return c2

                lax.fori_loop(0, JG, gbody, 0)

            out_copy(t).start()

            @pl.when(t + 2 < NCH)
            def _():
                idx_copy(t + 2).wait()
                for c in in_copies(t + 2):
                    c.start()

            @pl.when(t + NB < NCH)
            def _():
                idx_copy(t + NB).start()

            return carry

        lax.fori_loop(0, NCH, tbody, 0)
        out_copy(NCH - 2).wait()
        out_copy(NCH - 1).wait()
        pltpu.sync_copy(pout_all, pos_hbm.at[wid])

    return pl.kernel(
        body,
        out_type=[
            jax.ShapeDtypeStruct((NW, RW // 128, 128), jnp.float32),
            jax.ShapeDtypeStruct((N * NEG,), jnp.float32),
        ],
        mesh=mesh,
        compiler_params=pltpu.CompilerParams(
            needs_layout_passes=False, use_tc_tiling_on_sc=True),
        scratch_types=[
            pltpu.VMEM((NB * R * IW,), jnp.int32),
            pltpu.VMEM((NB, R, IW, 2 * D), jnp.float32),
            pltpu.VMEM((NB * R * D,), jnp.float32),
            pltpu.VMEM((NB * R * NEG,), jnp.float32),
            pltpu.VMEM((RW // 128, 128), jnp.float32),
            pltpu.SemaphoreType.DMA,
            pltpu.SemaphoreType.DMA,
            pltpu.SemaphoreType.DMA,
            pltpu.SemaphoreType.DMA,
        ],
    )


def _make_loss(N, NEG):
    rows = 2048
    inv_t = 1.0 / _TEMP

    def body(pos_ref, neg_ref, logits_ref, loss_ref):
        i = pl.program_id(0)
        pos = pos_ref[...]
        neg = neg_ref[...]
        logits_ref[...] = jnp.concatenate([pos, neg], axis=1)
        sp = pos * inv_t
        sn = neg * inv_t
        m = jnp.maximum(sp, jnp.max(sn, axis=1, keepdims=True))
        lse = m + jnp.log(
            jnp.exp(sp - m) + jnp.sum(jnp.exp(sn - m), axis=1, keepdims=True))
        c = jnp.sum(lse - sp)

        @pl.when(i == 0)
        def _():
            loss_ref[0, 0] = 0.0

        loss_ref[0, 0] += c / N

    return pl.pallas_call(
        body,
        grid=(N // rows,),
        in_specs=[
            pl.BlockSpec((rows, 1), lambda i: (i, 0)),
            pl.BlockSpec((rows, NEG), lambda i: (i, 0)),
        ],
        out_specs=[
            pl.BlockSpec((rows, 1 + NEG), lambda i: (i, 0)),
            pl.BlockSpec((1, 1), lambda i: (0, 0), memory_space=pltpu.SMEM),
        ],
        out_shape=[
            jax.ShapeDtypeStruct((N, 1 + NEG), jnp.float32),
            jax.ShapeDtypeStruct((1, 1), jnp.float32),
        ],
    )


def kernel(embeds, labels, table, W, b, neg_samples):
    B, T, D = embeds.shape
    N = B * T
    V = table.shape[0]
    NEG = neg_samples.shape[1]

    e2 = embeds.reshape(N, D)
    x = _project(e2, W.T, b.reshape(1, D), N, D)
    table_w = _widen_table(table.T, V, D)
    idxc = jnp.concatenate(
        [labels.reshape(N, 1), neg_samples,
         jnp.zeros((N, 7), jnp.int32)], axis=1).reshape(-1)
    pos3, neg1 = _make_scores(N, D, NEG)(x.reshape(-1), idxc, table_w)
    pos = pos3.reshape(N, 1)
    neg = neg1.reshape(N, NEG)
    logits, loss = _make_loss(N, NEG)(pos, neg)
    return logits, loss.reshape(())


# speedup vs baseline: 1.5115x; 1.0605x over previous
import jax
import jax.numpy as jnp
from jax import lax
from jax.experimental import pallas as pl
from jax.experimental.pallas import tpu as pltpu
from jax.experimental.pallas import tpu_sc as plsc

_TEMP = 1.0


def _proj_body(e_ref, wt_ref, b_ref, o_ref):
    o_ref[...] = (
        jnp.dot(e_ref[...], wt_ref[...], preferred_element_type=jnp.float32)
        + b_ref[...]
    )


def _project(e2, Wt, b2, N, D):
    rows = 2048
    return pl.pallas_call(
        _proj_body,
        grid=(N // rows,),
        in_specs=[
            pl.BlockSpec((rows, D), lambda i: (i, 0)),
            pl.BlockSpec((D, D), lambda i: (0, 0)),
            pl.BlockSpec((1, D), lambda i: (0, 0)),
        ],
        out_specs=pl.BlockSpec((rows, D), lambda i: (i, 0)),
        out_shape=jax.ShapeDtypeStruct((N, D), jnp.float32),
    )(e2, Wt, b2)


def _widen_body(t_ref, e_ref, o_ref):
    o_ref[:, 0:64] = lax.dot_general(
        t_ref[...], e_ref[...], (((0,), (0,)), ((), ())),
        preferred_element_type=jnp.float32)


def _widen_table(table_t, V, D):
    rows = 16384
    eye = jnp.eye(D, dtype=jnp.float32)
    return pl.pallas_call(
        _widen_body,
        grid=((V + rows - 1) // rows,),
        in_specs=[pl.BlockSpec((D, rows), lambda i: (0, i)),
                  pl.BlockSpec((D, D), lambda i: (0, 0))],
        out_specs=pl.BlockSpec((rows, 2 * D), lambda i: (i, 0)),
        out_shape=jax.ShapeDtypeStruct((V, 2 * D), jnp.float32),
    )(table_t, eye)


def _make_scores(N, D, NEG):
    NC, NS = 2, 16
    NW = NC * NS
    RW = N // NW
    R = 4
    NCH = RW // R
    L = 16
    JG = NEG // L
    IW = 72
    G = 1 + NEG
    mesh = plsc.VectorSubcoreMesh(
        core_axis_name="c", subcore_axis_name="s", num_cores=NC, num_subcores=NS
    )

    NB = 3

    def body(x_hbm, idxc_hbm, table_hbm, pos_hbm, neg_hbm,
             idx_v, comb_v, x_v, nout_v, pout_all,
             sem_idx, sem_in, sem_out, sem_spare):
        wid = lax.axis_index("s") * NC + lax.axis_index("c")
        base = wid * RW
        lane = jnp.arange(L, dtype=jnp.int32)

        def idx_copy(c):
            return pltpu.make_async_copy(
                idxc_hbm.at[pl.ds((base + c * R) * IW, R * IW)],
                idx_v.at[pl.ds((c % NB) * R * IW, R * IW)], sem_idx)

        def in_copies(c):
            b = c % NB
            cps = [
                pltpu.make_async_copy(
                    x_hbm.at[pl.ds((base + c * R) * D, R * D)],
                    x_v.at[pl.ds(b * R * D, R * D)], sem_in),
            ]
            for r in range(R):
                cps.append(pltpu.make_async_copy(
                    table_hbm.at[idx_v.at[pl.ds(b * R * IW + r * IW, G)]],
                    comb_v.at[b, r, pl.ds(0, G)], sem_in))
            return cps

        def out_copy(c):
            return pltpu.make_async_copy(
                nout_v.at[pl.ds((c % NB) * R * NEG, R * NEG)],
                neg_hbm.at[pl.ds((base + c * R) * NEG, R * NEG)], sem_out)

        for c in range(2):
            idx_copy(c).start()
            idx_copy(c).wait()
            for cp in in_copies(c):
                cp.start()
        idx_copy(2).start()

        def tbody(t, carry):
            b = t % NB
            bi = jnp.full((L,), b, jnp.int32)
            for c in in_copies(t):
                c.wait()

            @pl.when(t >= NB)
            def _():
                out_copy(t - NB).wait()

            @pl.when(t + 2 < NCH)
            def _():
                idx_copy(t + 2).wait()
                for c in in_copies(t + 2):
                    c.start()

            @pl.when(t + NB < NCH)
            def _():
                idx_copy(t + NB).start()

            for r in range(R):
                ri = jnp.full((L,), r, jnp.int32)

                def ld(row_vec, k):
                    return plsc.load_gather(
                        comb_v, [bi, ri, row_vec, lane + 16 * k])

                def xld(k, r=r):
                    return plsc.load_gather(
                        x_v, [bi * (R * D) + r * D + 16 * k + lane])

                x0, x1, x2, x3 = xld(0), xld(1), xld(2), xld(3)
                zero = jnp.zeros((L,), jnp.int32)
                ps = (ld(zero, 0) * x0 + ld(zero, 1) * x1
                      + ld(zero, 2) * x2 + ld(zero, 3) * x3)
                o = t * R + r
                plsc.store_scatter(
                    pout_all, [jnp.full((L,), o // 128, jnp.int32),
                               jnp.full((L,), o % 128, jnp.int32)],
                    jnp.full((L,), jnp.sum(ps), jnp.float32),
                    mask=lane == 0)

                def gbody(jg, c2, r=r, bi=bi, ri=ri,
                          x0=x0, x1=x1, x2=x2, x3=x3):
                    acc = jnp.zeros((L,), jnp.float32)
                    rb = jg * L + 1
                    for jj in range(L):
                        row_i = jnp.full((L,), rb + jj, jnp.int32)
                        a = (ld(row_i, 0) * x0 + ld(row_i, 1) * x1
                             + ld(row_i, 2) * x2 + ld(row_i, 3) * x3)
                        oh = jnp.where(lane == jj, 1.0, 0.0).astype(
                            jnp.float32)
                        acc = acc + jnp.sum(a) * oh
                    plsc.store_scatter(
                        nout_v,
                        [bi * (R * NEG) + r * NEG + jg * L + lane], acc)
                    return c2

                lax.fori_loop(0, JG, gbody, 0)

            out_copy(t).start()
            return carry

        lax.fori_loop(0, NCH, tbody, 0)
        out_copy(NCH - 2).wait()
        out_copy(NCH - 1).wait()
        pltpu.sync_copy(pout_all, pos_hbm.at[wid])

    return pl.kernel(
        body,
        out_type=[
            jax.ShapeDtypeStruct((NW, RW // 128, 128), jnp.float32),
            jax.ShapeDtypeStruct((N * NEG,), jnp.float32),
        ],
        mesh=mesh,
        compiler_params=pltpu.CompilerParams(
            needs_layout_passes=False, use_tc_tiling_on_sc=True),
        scratch_types=[
            pltpu.VMEM((NB * R * IW,), jnp.int32),
            pltpu.VMEM((NB, R, IW, 2 * D), jnp.float32),
            pltpu.VMEM((NB * R * D,), jnp.float32),
            pltpu.VMEM((NB * R * NEG,), jnp.float32),
            pltpu.VMEM((RW // 128, 128), jnp.float32),
            pltpu.SemaphoreType.DMA,
            pltpu.SemaphoreType.DMA,
            pltpu.SemaphoreType.DMA,
            pltpu.SemaphoreType.DMA,
        ],
    )


def _make_loss(N, NEG):
    rows = 2048
    inv_t = 1.0 / _TEMP

    def body(pos_ref, neg_ref, logits_ref, loss_ref):
        i = pl.program_id(0)
        pos = pos_ref[...]
        neg = neg_ref[...]
        logits_ref[...] = jnp.concatenate([pos, neg], axis=1)
        sp = pos * inv_t
        sn = neg * inv_t
        m = jnp.maximum(sp, jnp.max(sn, axis=1, keepdims=True))
        lse = m + jnp.log(
            jnp.exp(sp - m) + jnp.sum(jnp.exp(sn - m), axis=1, keepdims=True))
        c = jnp.sum(lse - sp)

        @pl.when(i == 0)
        def _():
            loss_ref[0, 0] = 0.0

        loss_ref[0, 0] += c / N

    return pl.pallas_call(
        body,
        grid=(N // rows,),
        in_specs=[
            pl.BlockSpec((rows, 1), lambda i: (i, 0)),
            pl.BlockSpec((rows, NEG), lambda i: (i, 0)),
        ],
        out_specs=[
            pl.BlockSpec((rows, 1 + NEG), lambda i: (i, 0)),
            pl.BlockSpec((1, 1), lambda i: (0, 0), memory_space=pltpu.SMEM),
        ],
        out_shape=[
            jax.ShapeDtypeStruct((N, 1 + NEG), jnp.float32),
            jax.ShapeDtypeStruct((1, 1), jnp.float32),
        ],
    )


def kernel(embeds, labels, table, W, b, neg_samples):
    B, T, D = embeds.shape
    N = B * T
    V = table.shape[0]
    NEG = neg_samples.shape[1]

    e2 = embeds.reshape(N, D)
    x = _project(e2, W.T, b.reshape(1, D), N, D)
    table_w = _widen_table(table.T, V, D)
    idxc = jnp.concatenate(
        [labels.reshape(N, 1), neg_samples,
         jnp.zeros((N, 7), jnp.int32)], axis=1).reshape(-1)
    pos3, neg1 = _make_scores(N, D, NEG)(x.reshape(-1), idxc, table_w)
    pos = pos3.reshape(N, 1)
    neg = neg1.reshape(N, NEG)
    logits, loss = _make_loss(N, NEG)(pos, neg)
    return logits, loss.reshape(())


# speedup vs baseline: 1.5262x; 1.0097x over previous
import jax
import jax.numpy as jnp
from jax import lax
from jax.experimental import pallas as pl
from jax.experimental.pallas import tpu as pltpu
from jax.experimental.pallas import tpu_sc as plsc

_TEMP = 1.0


def _proj_body(e_ref, wt_ref, b_ref, o_ref):
    o_ref[...] = (
        jnp.dot(e_ref[...], wt_ref[...], preferred_element_type=jnp.float32)
        + b_ref[...]
    )


def _project(e2, Wt, b2, N, D):
    rows = 2048
    return pl.pallas_call(
        _proj_body,
        grid=(N // rows,),
        in_specs=[
            pl.BlockSpec((rows, D), lambda i: (i, 0)),
            pl.BlockSpec((D, D), lambda i: (0, 0)),
            pl.BlockSpec((1, D), lambda i: (0, 0)),
        ],
        out_specs=pl.BlockSpec((rows, D), lambda i: (i, 0)),
        out_shape=jax.ShapeDtypeStruct((N, D), jnp.float32),
    )(e2, Wt, b2)


def _widen_body(t_ref, e_ref, o_ref):
    o_ref[:, 0:64] = lax.dot_general(
        t_ref[...], e_ref[...], (((0,), (0,)), ((), ())),
        preferred_element_type=jnp.float32)


def _widen_table(table_t, V, D):
    rows = 32768
    eye = jnp.eye(D, dtype=jnp.float32)
    return pl.pallas_call(
        _widen_body,
        grid=((V + rows - 1) // rows,),
        in_specs=[pl.BlockSpec((D, rows), lambda i: (0, i)),
                  pl.BlockSpec((D, D), lambda i: (0, 0))],
        out_specs=pl.BlockSpec((rows, 2 * D), lambda i: (i, 0)),
        out_shape=jax.ShapeDtypeStruct((V, 2 * D), jnp.float32),
        compiler_params=pltpu.CompilerParams(
            vmem_limit_bytes=100 * 1024 * 1024),
    )(table_t, eye)


def _make_scores(N, D, NEG):
    NC, NS = 2, 16
    NW = NC * NS
    RW = N // NW
    R = 4
    NCH = RW // R
    L = 16
    JG = NEG // L
    IW = 72
    G = 1 + NEG
    mesh = plsc.VectorSubcoreMesh(
        core_axis_name="c", subcore_axis_name="s", num_cores=NC, num_subcores=NS
    )

    NB = 3

    def body(x_hbm, idxc_hbm, table_hbm, pos_hbm, neg_hbm,
             idx_v, comb_v, x_v, nout_v, pout_all,
             sem_idx, sem_in, sem_out, sem_spare):
        wid = lax.axis_index("s") * NC + lax.axis_index("c")
        base = wid * RW
        lane = jnp.arange(L, dtype=jnp.int32)

        def idx_copy(c):
            return pltpu.make_async_copy(
                idxc_hbm.at[pl.ds((base + c * R) * IW, R * IW)],
                idx_v.at[pl.ds((c % NB) * R * IW, R * IW)], sem_idx)

        def in_copies(c):
            b = c % NB
            cps = [
                pltpu.make_async_copy(
                    x_hbm.at[pl.ds((base + c * R) * D, R * D)],
                    x_v.at[pl.ds(b * R * D, R * D)], sem_in),
            ]
            for r in range(R):
                cps.append(pltpu.make_async_copy(
                    table_hbm.at[idx_v.at[pl.ds(b * R * IW + r * IW, G)]],
                    comb_v.at[b, r, pl.ds(0, G)], sem_in))
            return cps

        def out_copy(c):
            return pltpu.make_async_copy(
                nout_v.at[pl.ds((c % NB) * R * NEG, R * NEG)],
                neg_hbm.at[pl.ds((base + c * R) * NEG, R * NEG)], sem_out)

        for c in range(2):
            idx_copy(c).start()
            idx_copy(c).wait()
            for cp in in_copies(c):
                cp.start()
        idx_copy(2).start()

        def tbody(t, carry):
            b = t % NB
            bi = jnp.full((L,), b, jnp.int32)
            for c in in_copies(t):
                c.wait()

            @pl.when(t >= NB)
            def _():
                out_copy(t - NB).wait()

            @pl.when(t + 2 < NCH)
            def _():
                idx_copy(t + 2).wait()
                for c in in_copies(t + 2):
                    c.start()

            @pl.when(t + NB < NCH)
            def _():
                idx_copy(t + NB).start()

            for r in range(R):
                ri = jnp.full((L,), r, jnp.int32)

                def ld(row_vec, k):
                    return plsc.load_gather(
                        comb_v, [bi, ri, row_vec, lane + 16 * k])

                def xld(k, r=r):
                    return plsc.load_gather(
                        x_v, [bi * (R * D) + r * D + 16 * k + lane])

                x0, x1, x2, x3 = xld(0), xld(1), xld(2), xld(3)
                zero = jnp.zeros((L,), jnp.int32)
                ps = (ld(zero, 0) * x0 + ld(zero, 1) * x1
                      + ld(zero, 2) * x2 + ld(zero, 3) * x3)
                o = t * R + r
                plsc.store_scatter(
                    pout_all, [jnp.full((L,), o // 128, jnp.int32),
                               jnp.full((L,), o % 128, jnp.int32)],
                    jnp.full((L,), jnp.sum(ps), jnp.float32),
                    mask=lane == 0)

                def gbody(jg, c2, r=r, bi=bi, ri=ri,
                          x0=x0, x1=x1, x2=x2, x3=x3):
                    acc = jnp.zeros((L,), jnp.float32)
                    rb = jg * L + 1
                    for jj in range(L):
                        row_i = jnp.full((L,), rb + jj, jnp.int32)
                        a = (ld(row_i, 0) * x0 + ld(row_i, 1) * x1
                             + ld(row_i, 2) * x2 + ld(row_i, 3) * x3)
                        oh = jnp.where(lane == jj, 1.0, 0.0).astype(
                            jnp.float32)
                        acc = acc + jnp.sum(a) * oh
                    plsc.store_scatter(
                        nout_v,
                        [bi * (R * NEG) + r * NEG + jg * L + lane], acc)
                    return c2

                lax.fori_loop(0, JG, gbody, 0)

            out_copy(t).start()
            return carry

        lax.fori_loop(0, NCH, tbody, 0)
        out_copy(NCH - 2).wait()
        out_copy(NCH - 1).wait()
        pltpu.sync_copy(pout_all, pos_hbm.at[wid])

    return pl.kernel(
        body,
        out_type=[
            jax.ShapeDtypeStruct((NW, RW // 128, 128), jnp.float32),
            jax.ShapeDtypeStruct((N * NEG,), jnp.float32),
        ],
        mesh=mesh,
        compiler_params=pltpu.CompilerParams(
            needs_layout_passes=False, use_tc_tiling_on_sc=True),
        scratch_types=[
            pltpu.VMEM((NB * R * IW,), jnp.int32),
            pltpu.VMEM((NB, R, IW, 2 * D), jnp.float32),
            pltpu.VMEM((NB * R * D,), jnp.float32),
            pltpu.VMEM((NB * R * NEG,), jnp.float32),
            pltpu.VMEM((RW // 128, 128), jnp.float32),
            pltpu.SemaphoreType.DMA,
            pltpu.SemaphoreType.DMA,
            pltpu.SemaphoreType.DMA,
            pltpu.SemaphoreType.DMA,
        ],
    )


def _make_loss(N, NEG):
    rows = 2048
    inv_t = 1.0 / _TEMP

    def body(pos_ref, neg_ref, logits_ref, loss_ref):
        i = pl.program_id(0)
        pos = pos_ref[...]
        neg = neg_ref[...]
        logits_ref[...] = jnp.concatenate([pos, neg], axis=1)
        sp = pos * inv_t
        sn = neg * inv_t
        m = jnp.maximum(sp, jnp.max(sn, axis=1, keepdims=True))
        lse = m + jnp.log(
            jnp.exp(sp - m) + jnp.sum(jnp.exp(sn - m), axis=1, keepdims=True))
        c = jnp.sum(lse - sp)

        @pl.when(i == 0)
        def _():
            loss_ref[0, 0] = 0.0

        loss_ref[0, 0] += c / N

    return pl.pallas_call(
        body,
        grid=(N // rows,),
        in_specs=[
            pl.BlockSpec((rows, 1), lambda i: (i, 0)),
            pl.BlockSpec((rows, NEG), lambda i: (i, 0)),
        ],
        out_specs=[
            pl.BlockSpec((rows, 1 + NEG), lambda i: (i, 0)),
            pl.BlockSpec((1, 1), lambda i: (0, 0), memory_space=pltpu.SMEM),
        ],
        out_shape=[
            jax.ShapeDtypeStruct((N, 1 + NEG), jnp.float32),
            jax.ShapeDtypeStruct((1, 1), jnp.float32),
        ],
    )


def kernel(embeds, labels, table, W, b, neg_samples):
    B, T, D = embeds.shape
    N = B * T
    V = table.shape[0]
    NEG = neg_samples.shape[1]

    e2 = embeds.reshape(N, D)
    x = _project(e2, W.T, b.reshape(1, D), N, D)
    table_w = _widen_table(table.T, V, D)
    idxc = jnp.concatenate(
        [labels.reshape(N, 1), neg_samples,
         jnp.zeros((N, 7), jnp.int32)], axis=1).reshape(-1)
    pos3, neg1 = _make_scores(N, D, NEG)(x.reshape(-1), idxc, table_w)
    pos = pos3.reshape(N, 1)
    neg = neg1.reshape(N, NEG)
    logits, loss = _make_loss(N, NEG)(pos, neg)
    return logits, loss.reshape(())
